# baseline (speedup 1.0000x reference)
import functools

import jax
import jax.numpy as jnp
from jax.experimental import pallas as pl
from jax.experimental.pallas import tpu as pltpu
from jax.experimental.pallas import tpu_sc as plsc

HID = 256
N_NODES = 10000
E = 160000
WIN = 128
E_PAD = 163840
NODE_BLK = 1000
DEC_BLK = 2048

_MESH = plsc.VectorSubcoreMesh(core_axis_name="c", subcore_axis_name="s")


def _pack_bf16_words(x):
    u = jax.lax.bitcast_convert_type(x, jnp.uint32)
    r = (u + jnp.uint32(0x8000)) >> 16
    word = r[:, :HID // 2] | (r[:, HID // 2:] << 16)
    return jax.lax.bitcast_convert_type(word, jnp.int32)


def _mlp1_body(z_ref, w1a_ref, w1b_ref, b1_ref, za_ref, zb_ref):
    z = z_ref[...]
    za = jnp.dot(z, w1a_ref[...], preferred_element_type=jnp.float32)
    zb = jnp.dot(z, w1b_ref[...], preferred_element_type=jnp.float32)
    za_ref[...] = _pack_bf16_words(za + b1_ref[...])
    zb_ref[...] = _pack_bf16_words(zb)


def _precompute_tables(z, w1a, w1b, b1row):
    return pl.pallas_call(
        _mlp1_body,
        grid=(N_NODES // NODE_BLK,),
        in_specs=[
            pl.BlockSpec((NODE_BLK, HID), lambda i: (i, 0)),
            pl.BlockSpec((HID, HID), lambda i: (0, 0)),
            pl.BlockSpec((HID, HID), lambda i: (0, 0)),
            pl.BlockSpec((1, HID), lambda i: (0, 0)),
        ],
        out_specs=[
            pl.BlockSpec((NODE_BLK, HID // 2), lambda i: (i, 0)),
            pl.BlockSpec((NODE_BLK, HID // 2), lambda i: (i, 0)),
        ],
        out_shape=[
            jax.ShapeDtypeStruct((N_NODES, HID // 2), jnp.int32),
            jax.ShapeDtypeStruct((N_NODES, HID // 2), jnp.int32),
        ],
    )(z, w1a, w1b, b1row)


N_WORKERS = 32
N_CHUNKS = 1


def _sc_gather(row_p, col_p, za, zb):
    ec = row_p.shape[1]
    eps = ec // N_WORKERS
    nwin = eps // WIN
    from jax import lax

    D = 2

    @functools.partial(
        pl.kernel,
        out_type=(
            jax.ShapeDtypeStruct((ec, HID // 2), jnp.int32),
            jax.ShapeDtypeStruct((ec, HID // 2), jnp.int32),
        ),
        mesh=_MESH,
        scratch_types=(
            [pltpu.VMEM((eps,), jnp.int32)] * 2
            + [pltpu.VMEM((WIN, HID // 2), jnp.int32)] * (2 * D)
            + [pltpu.SemaphoreType.DMA] * (1 + 2 * D)
        ),
    )
    def k(row_hbm, col_hbm, za_hbm, zb_hbm, ga_hbm, gb_hbm,
          idxr_v, idxc_v, gav0, gbv0, gav1, gbv1,
          isem, gsem0, gsem1, wsem0, wsem1):
        ga_v, gb_v = [gav0, gav1], [gbv0, gbv1]
        gsem, wsem = [gsem0, gsem1], [wsem0, wsem1]
        wid = lax.axis_index("s") * 2 + lax.axis_index("c")
        base = wid * eps
        ci = pltpu.async_copy(row_hbm.at[0, pl.ds(base, eps)], idxr_v, isem)
        cc = pltpu.async_copy(col_hbm.at[0, pl.ds(base, eps)], idxc_v, isem)
        ci.wait()
        cc.wait()

        def issue_gather(w, b):
            off = w * WIN
            pltpu.async_copy(
                za_hbm.at[idxr_v.at[pl.ds(off, WIN)]], ga_v[b], gsem[b])
            pltpu.async_copy(
                zb_hbm.at[idxc_v.at[pl.ds(off, WIN)]], gb_v[b], gsem[b])

        def wait_gather(b):
            pltpu.make_async_copy(
                za_hbm.at[pl.ds(0, WIN)], ga_v[b], gsem[b]).wait()
            pltpu.make_async_copy(
                zb_hbm.at[pl.ds(0, WIN)], gb_v[b], gsem[b]).wait()

        def issue_wo(w, b):
            off = base + w * WIN
            pltpu.async_copy(ga_v[b], ga_hbm.at[pl.ds(off, WIN)], wsem[b])
            pltpu.async_copy(gb_v[b], gb_hbm.at[pl.ds(off, WIN)], wsem[b])

        def wait_wo(b):
            pltpu.make_async_copy(
                ga_v[b], ga_hbm.at[pl.ds(0, WIN)], wsem[b]).wait()
            pltpu.make_async_copy(
                gb_v[b], gb_hbm.at[pl.ds(0, WIN)], wsem[b]).wait()

        for b in range(D):
            issue_gather(b, b)

        @pl.loop(0, nwin // D - 1)
        def _(g):
            w0 = g * D
            for b in range(D):
                wait_gather(b)
                issue_wo(w0 + b, b)
                wait_wo(b)
                issue_gather(w0 + b + D, b)

        for b in range(D):
            wait_gather(b)
            issue_wo(nwin - D + b, b)
            wait_wo(b)

    return k(row_p, col_p, za, zb)


def _unpack_pair(x_i32):
    lo = jax.lax.bitcast_convert_type(x_i32 << 16, jnp.float32)
    hi = jax.lax.bitcast_convert_type(x_i32 & jnp.int32(-65536), jnp.float32)
    return lo, hi


def _decode_body(ga_ref, gb_ref, w2m_ref, b2_ref, o_ref):
    ga_lo, ga_hi = _unpack_pair(ga_ref[...])
    gb_lo, gb_hi = _unpack_pair(gb_ref[...])
    h_lo = jnp.maximum(ga_lo + gb_lo, 0.0).astype(jnp.bfloat16)
    h_hi = jnp.maximum(ga_hi + gb_hi, 0.0).astype(jnp.bfloat16)
    h = jnp.concatenate([h_lo, h_hi], axis=1)
    acc = jnp.dot(h, w2m_ref[...], preferred_element_type=jnp.float32)
    logit = acc[:, 0] + b2_ref[0, 0]
    o_ref[...] = 1.0 / (1.0 + jnp.exp(-logit))


def _decode(ga, gb, w2m, b2):
    ec = ga.shape[0]
    return pl.pallas_call(
        _decode_body,
        grid=(ec // DEC_BLK,),
        in_specs=[
            pl.BlockSpec((DEC_BLK, HID // 2), lambda i: (i, 0)),
            pl.BlockSpec((DEC_BLK, HID // 2), lambda i: (i, 0)),
            pl.BlockSpec((HID, 128), lambda i: (0, 0)),
            pl.BlockSpec((1, 1), lambda i: (0, 0)),
        ],
        out_specs=pl.BlockSpec((DEC_BLK,), lambda i: (i,)),
        out_shape=jax.ShapeDtypeStruct((ec,), jnp.float32),
    )(ga, gb, w2m, b2)


def kernel(z, edge_index, W1, b1, W2, b2):
    row = edge_index[0].astype(jnp.int32)
    col = edge_index[1].astype(jnp.int32)
    pad = jnp.zeros((E_PAD - E,), jnp.int32)
    row_p = jnp.concatenate([row, pad]).reshape(1, E_PAD)
    col_p = jnp.concatenate([col, pad]).reshape(1, E_PAD)
    w1a = W1[:, :HID].T
    w1b = W1[:, HID:].T
    za_p, zb_p = _precompute_tables(z, w1a, w1b, b1.reshape(1, HID))
    w2m = jnp.zeros((HID, 128), jnp.bfloat16)
    w2m = w2m.at[:, 0].set(W2[0].astype(jnp.bfloat16))
    b2r = b2.reshape(1, 1)
    ec = E_PAD // N_CHUNKS
    outs = []
    for c in range(N_CHUNKS):
        ga, gb = _sc_gather(row_p[:, c * ec:(c + 1) * ec],
                            col_p[:, c * ec:(c + 1) * ec], za_p, zb_p)
        outs.append(_decode(ga, gb, w2m, b2r))
    out_p = jnp.concatenate(outs)
    return out_p[:E]


# speedup vs baseline: 1.1595x; 1.1595x over previous
import functools

import jax
import jax.numpy as jnp
from jax.experimental import pallas as pl
from jax.experimental.pallas import tpu as pltpu
from jax.experimental.pallas import tpu_sc as plsc

HID = 256
N_NODES = 10000
E = 160000
WIN = 128
E_PAD = 163840
NODE_BLK = 1000
DEC_BLK = 2048

_MESH = plsc.VectorSubcoreMesh(core_axis_name="c", subcore_axis_name="s")


def _pack_bf16_words(x):
    u = jax.lax.bitcast_convert_type(x, jnp.uint32)
    r = (u + jnp.uint32(0x8000)) >> 16
    word = r[:, :HID // 2] | (r[:, HID // 2:] << 16)
    return jax.lax.bitcast_convert_type(word, jnp.int32)


def _mlp1_body(z_ref, w1a_ref, w1b_ref, b1_ref, za_ref, zb_ref):
    z = z_ref[...]
    za = jnp.dot(z, w1a_ref[...], preferred_element_type=jnp.float32)
    zb = jnp.dot(z, w1b_ref[...], preferred_element_type=jnp.float32)
    za_ref[...] = _pack_bf16_words(za + b1_ref[...])
    zb_ref[...] = _pack_bf16_words(zb)


def _precompute_tables(z, w1a, w1b, b1row):
    return pl.pallas_call(
        _mlp1_body,
        grid=(N_NODES // NODE_BLK,),
        in_specs=[
            pl.BlockSpec((NODE_BLK, HID), lambda i: (i, 0)),
            pl.BlockSpec((HID, HID), lambda i: (0, 0)),
            pl.BlockSpec((HID, HID), lambda i: (0, 0)),
            pl.BlockSpec((1, HID), lambda i: (0, 0)),
        ],
        out_specs=[
            pl.BlockSpec((NODE_BLK, HID // 2), lambda i: (i, 0)),
            pl.BlockSpec((NODE_BLK, HID // 2), lambda i: (i, 0)),
        ],
        out_shape=[
            jax.ShapeDtypeStruct((N_NODES, HID // 2), jnp.int32),
            jax.ShapeDtypeStruct((N_NODES, HID // 2), jnp.int32),
        ],
    )(z, w1a, w1b, b1row)


N_WORKERS = 32
N_CHUNKS = 4


def _sc_gather(row_p, col_p, za, zb):
    ec = row_p.shape[1]
    eps = ec // N_WORKERS
    nwin = eps // WIN
    from jax import lax

    D = 2

    @functools.partial(
        pl.kernel,
        out_type=(
            jax.ShapeDtypeStruct((ec, HID // 2), jnp.int32),
            jax.ShapeDtypeStruct((ec, HID // 2), jnp.int32),
        ),
        mesh=_MESH,
        scratch_types=(
            [pltpu.VMEM((eps,), jnp.int32)] * 2
            + [pltpu.VMEM((WIN, HID // 2), jnp.int32)] * (2 * D)
            + [pltpu.SemaphoreType.DMA] * (1 + 2 * D)
        ),
    )
    def k(row_hbm, col_hbm, za_hbm, zb_hbm, ga_hbm, gb_hbm,
          idxr_v, idxc_v, gav0, gbv0, gav1, gbv1,
          isem, gsem0, gsem1, wsem0, wsem1):
        ga_v, gb_v = [gav0, gav1], [gbv0, gbv1]
        gsem, wsem = [gsem0, gsem1], [wsem0, wsem1]
        wid = lax.axis_index("s") * 2 + lax.axis_index("c")
        base = wid * eps
        ci = pltpu.async_copy(row_hbm.at[0, pl.ds(base, eps)], idxr_v, isem)
        cc = pltpu.async_copy(col_hbm.at[0, pl.ds(base, eps)], idxc_v, isem)
        ci.wait()
        cc.wait()

        def issue_gather(w, b):
            off = w * WIN
            pltpu.async_copy(
                za_hbm.at[idxr_v.at[pl.ds(off, WIN)]], ga_v[b], gsem[b])
            pltpu.async_copy(
                zb_hbm.at[idxc_v.at[pl.ds(off, WIN)]], gb_v[b], gsem[b])

        def wait_gather(b):
            pltpu.make_async_copy(
                za_hbm.at[pl.ds(0, WIN)], ga_v[b], gsem[b]).wait()
            pltpu.make_async_copy(
                zb_hbm.at[pl.ds(0, WIN)], gb_v[b], gsem[b]).wait()

        def issue_wo(w, b):
            off = base + w * WIN
            pltpu.async_copy(ga_v[b], ga_hbm.at[pl.ds(off, WIN)], wsem[b])
            pltpu.async_copy(gb_v[b], gb_hbm.at[pl.ds(off, WIN)], wsem[b])

        def wait_wo(b):
            pltpu.make_async_copy(
                ga_v[b], ga_hbm.at[pl.ds(0, WIN)], wsem[b]).wait()
            pltpu.make_async_copy(
                gb_v[b], gb_hbm.at[pl.ds(0, WIN)], wsem[b]).wait()

        for b in range(D):
            issue_gather(b, b)

        @pl.loop(0, nwin // D - 1)
        def _(g):
            w0 = g * D
            for b in range(D):
                wait_gather(b)
                issue_wo(w0 + b, b)
                wait_wo(b)
                issue_gather(w0 + b + D, b)

        for b in range(D):
            wait_gather(b)
            issue_wo(nwin - D + b, b)
            wait_wo(b)

    return k(row_p, col_p, za, zb)


def _unpack_pair(x_i32):
    lo = jax.lax.bitcast_convert_type(x_i32 << 16, jnp.float32)
    hi = jax.lax.bitcast_convert_type(x_i32 & jnp.int32(-65536), jnp.float32)
    return lo, hi


def _decode_body(ga_ref, gb_ref, w2m_ref, b2_ref, o_ref):
    ga_lo, ga_hi = _unpack_pair(ga_ref[...])
    gb_lo, gb_hi = _unpack_pair(gb_ref[...])
    h_lo = jnp.maximum(ga_lo + gb_lo, 0.0).astype(jnp.bfloat16)
    h_hi = jnp.maximum(ga_hi + gb_hi, 0.0).astype(jnp.bfloat16)
    h = jnp.concatenate([h_lo, h_hi], axis=1)
    acc = jnp.dot(h, w2m_ref[...], preferred_element_type=jnp.float32)
    logit = acc[:, 0] + b2_ref[0, 0]
    o_ref[...] = 1.0 / (1.0 + jnp.exp(-logit))


def _decode(ga, gb, w2m, b2):
    ec = ga.shape[0]
    return pl.pallas_call(
        _decode_body,
        grid=(ec // DEC_BLK,),
        in_specs=[
            pl.BlockSpec((DEC_BLK, HID // 2), lambda i: (i, 0)),
            pl.BlockSpec((DEC_BLK, HID // 2), lambda i: (i, 0)),
            pl.BlockSpec((HID, 128), lambda i: (0, 0)),
            pl.BlockSpec((1, 1), lambda i: (0, 0)),
        ],
        out_specs=pl.BlockSpec((DEC_BLK,), lambda i: (i,)),
        out_shape=jax.ShapeDtypeStruct((ec,), jnp.float32),
    )(ga, gb, w2m, b2)


def kernel(z, edge_index, W1, b1, W2, b2):
    row = edge_index[0].astype(jnp.int32)
    col = edge_index[1].astype(jnp.int32)
    pad = jnp.zeros((E_PAD - E,), jnp.int32)
    row_p = jnp.concatenate([row, pad]).reshape(1, E_PAD)
    col_p = jnp.concatenate([col, pad]).reshape(1, E_PAD)
    w1a = W1[:, :HID].T
    w1b = W1[:, HID:].T
    za_p, zb_p = _precompute_tables(z, w1a, w1b, b1.reshape(1, HID))
    w2m = jnp.zeros((HID, 128), jnp.bfloat16)
    w2m = w2m.at[:, 0].set(W2[0].astype(jnp.bfloat16))
    b2r = b2.reshape(1, 1)
    ec = E_PAD // N_CHUNKS
    outs = []
    for c in range(N_CHUNKS):
        ga, gb = _sc_gather(row_p[:, c * ec:(c + 1) * ec],
                            col_p[:, c * ec:(c + 1) * ec], za_p, zb_p)
        outs.append(_decode(ga, gb, w2m, b2r))
    out_p = jnp.concatenate(outs)
    return out_p[:E]
